# baseline (speedup 1.0000x reference)
import functools

import jax
import jax.numpy as jnp
from jax import lax
from jax.experimental import pallas as pl
from jax.experimental.pallas import tpu as pltpu
from jax.experimental.pallas import tpu_sc as plsc

_B, _L, _V, _H = 4096, 200, 1000000, 64
_NC, _NS = 2, 16
_NW = _NC * _NS
_BPW = _B // _NW
_C0 = 128
_C1 = _L - _C0

_mesh = plsc.VectorSubcoreMesh(core_axis_name="c", subcore_axis_name="s")


@functools.partial(
    pl.kernel,
    out_type=jax.ShapeDtypeStruct((_B, _H), jnp.float32),
    mesh=_mesh,
    scratch_types=[
        pltpu.VMEM((_BPW, _L), jnp.int32),
        pltpu.VMEM((_L, _H), jnp.float32),
        pltpu.VMEM((_BPW, _H), jnp.float32),
        pltpu.SemaphoreType.DMA,
    ],
    compiler_params=pltpu.CompilerParams(use_tc_tiling_on_sc=False),
)
def _bow_sum_sc(x_hbm, table_hbm, out_hbm, idx_v, rows_v, acc_v, sem):
    wid = lax.axis_index("s") * _NC + lax.axis_index("c")
    base = wid * _BPW
    pltpu.sync_copy(x_hbm.at[pl.ds(base, _BPW), :], idx_v)

    @pl.loop(0, _BPW)
    def _row(b):
        cp0 = pltpu.async_copy(
            table_hbm.at[idx_v.at[b, pl.ds(0, _C0)]],
            rows_v.at[pl.ds(0, _C0), :], sem)
        cp1 = pltpu.async_copy(
            table_hbm.at[idx_v.at[b, pl.ds(_C0, _C1)]],
            rows_v.at[pl.ds(_C0, _C1), :], sem)
        cp0.wait()
        cp1.wait()

        def ibody(l, accs):
            a0, a1, a2, a3 = accs
            return (a0 + rows_v[l, pl.ds(0, 16)],
                    a1 + rows_v[l, pl.ds(16, 16)],
                    a2 + rows_v[l, pl.ds(32, 16)],
                    a3 + rows_v[l, pl.ds(48, 16)])

        z = jnp.zeros((16,), jnp.float32)
        a0, a1, a2, a3 = lax.fori_loop(0, _L, ibody, (z, z, z, z))
        acc_v[b, pl.ds(0, 16)] = a0
        acc_v[b, pl.ds(16, 16)] = a1
        acc_v[b, pl.ds(32, 16)] = a2
        acc_v[b, pl.ds(48, 16)] = a3

    pltpu.sync_copy(acc_v, out_hbm.at[pl.ds(base, _BPW), :])


def _bn_relu(h, g, be, eps=1e-5):
    mu = jnp.mean(h, axis=0, keepdims=True)
    var = jnp.mean((h - mu) * (h - mu), axis=0, keepdims=True)
    return jnp.maximum(g * (h - mu) / jnp.sqrt(var + eps) + be, 0.0)


def _mlp_body(bow_ref, t_ref, W1, b1, g1, be1, W2, b2, g2, be2,
              W3, b3, g3, be3, Wout, bout, loss_ref, logits_ref):
    bow = bow_ref[...] * (1.0 / _L)
    h = jnp.dot(bow, W1[...], preferred_element_type=jnp.float32) + b1[...]
    h = _bn_relu(h, g1[...], be1[...])
    h = jnp.dot(h, W2[...], preferred_element_type=jnp.float32) + b2[...]
    h = _bn_relu(h, g2[...], be2[...])
    h = jnp.dot(h, W3[...], preferred_element_type=jnp.float32) + b3[...]
    h = _bn_relu(h, g3[...], be3[...])
    logits = jnp.dot(h, Wout[...], preferred_element_type=jnp.float32) + bout[...]
    logits_ref[...] = logits
    t = t_ref[...]
    per = (jnp.maximum(logits, 0.0) - logits * t
           + jnp.log1p(jnp.exp(-jnp.abs(logits))))
    loss_ref[...] = jnp.mean(per).reshape(1, 1)


_mlp = pl.pallas_call(
    _mlp_body,
    out_shape=(jax.ShapeDtypeStruct((1, 1), jnp.float32),
               jax.ShapeDtypeStruct((_B, 1), jnp.float32)),
)


def kernel(x, t, table, W1, b1, g1, be1, W2, b2, g2, be2,
           W3, b3, g3, be3, Wout, bout):
    xi = x.astype(jnp.int32)
    bow_sum = _bow_sum_sc(xi, table)
    loss2, logits2 = _mlp(
        bow_sum, t.reshape(_B, 1),
        W1, b1.reshape(1, _H), g1.reshape(1, _H), be1.reshape(1, _H),
        W2, b2.reshape(1, _H), g2.reshape(1, _H), be2.reshape(1, _H),
        W3, b3.reshape(1, _H), g3.reshape(1, _H), be3.reshape(1, _H),
        Wout, bout.reshape(1, 1))
    return loss2[0, 0], logits2[:, 0]


# speedup vs baseline: 1.1065x; 1.1065x over previous
import functools

import jax
import jax.numpy as jnp
from jax import lax
from jax.experimental import pallas as pl
from jax.experimental.pallas import tpu as pltpu
from jax.experimental.pallas import tpu_sc as plsc

_B, _L, _V, _H = 4096, 200, 1000000, 64
_NC, _NS = 2, 16
_NW = _NC * _NS
_BPW = _B // _NW
_C0 = 128
_C1 = _L - _C0
_UN = 8

_mesh = plsc.VectorSubcoreMesh(core_axis_name="c", subcore_axis_name="s")


@functools.partial(
    pl.kernel,
    out_type=jax.ShapeDtypeStruct((_B, _H), jnp.float32),
    mesh=_mesh,
    scratch_types=[
        pltpu.VMEM((_BPW, _L), jnp.int32),
        pltpu.VMEM((2, _L, _H), jnp.float32),
        pltpu.VMEM((_BPW, _H), jnp.float32),
        pltpu.SemaphoreType.DMA,
        pltpu.SemaphoreType.DMA,
    ],
    compiler_params=pltpu.CompilerParams(use_tc_tiling_on_sc=False),
)
def _bow_sum_sc(x_hbm, table_hbm, out_hbm, idx_v, rows_v, acc_v, s0, s1):
    wid = lax.axis_index("s") * _NC + lax.axis_index("c")
    base = wid * _BPW
    sems = (s0, s1)
    zero = jnp.zeros((16,), jnp.float32)
    pltpu.sync_copy(x_hbm.at[pl.ds(base, _BPW), :], idx_v)

    def fire(b, rslot):
        sem = sems[rslot]
        pltpu.async_copy(table_hbm.at[idx_v.at[b, pl.ds(0, _C0)]],
                         rows_v.at[rslot, pl.ds(0, _C0), :], sem)
        pltpu.async_copy(table_hbm.at[idx_v.at[b, pl.ds(_C0, _C1)]],
                         rows_v.at[rslot, pl.ds(_C0, _C1), :], sem)

    def drain(rslot):
        pltpu.make_async_copy(table_hbm.at[pl.ds(0, _L), :],
                              rows_v.at[rslot], sems[rslot]).wait()

    def accum_store(aslot, arow):
        def body(l, accs):
            a0, a1, a2, a3 = accs
            return (a0 + rows_v[aslot, l, pl.ds(0, 16)],
                    a1 + rows_v[aslot, l, pl.ds(16, 16)],
                    a2 + rows_v[aslot, l, pl.ds(32, 16)],
                    a3 + rows_v[aslot, l, pl.ds(48, 16)])

        a0, a1, a2, a3 = lax.fori_loop(0, _L, body, (zero,) * 4, unroll=_UN)
        acc_v[arow, pl.ds(0, 16)] = a0
        acc_v[arow, pl.ds(16, 16)] = a1
        acc_v[arow, pl.ds(32, 16)] = a2
        acc_v[arow, pl.ds(48, 16)] = a3

    fire(0, 0)

    @pl.loop(0, _BPW - 2, step=2)
    def _pair(b):
        drain(0)
        fire(b + 1, 1)
        accum_store(0, b)
        drain(1)
        fire(b + 2, 0)
        accum_store(1, b + 1)

    drain(0)
    fire(_BPW - 1, 1)
    accum_store(0, _BPW - 2)
    drain(1)
    accum_store(1, _BPW - 1)
    pltpu.sync_copy(acc_v, out_hbm.at[pl.ds(base, _BPW), :])


def _bn_relu(h, g, be, eps=1e-5):
    mu = jnp.mean(h, axis=0, keepdims=True)
    var = jnp.mean((h - mu) * (h - mu), axis=0, keepdims=True)
    return jnp.maximum(g * (h - mu) / jnp.sqrt(var + eps) + be, 0.0)


def _mlp_body(bow_ref, t_ref, W1, b1, g1, be1, W2, b2, g2, be2,
              W3, b3, g3, be3, Wout, bout, loss_ref, logits_ref):
    bow = bow_ref[...] * (1.0 / _L)
    h = jnp.dot(bow, W1[...], preferred_element_type=jnp.float32) + b1[...]
    h = _bn_relu(h, g1[...], be1[...])
    h = jnp.dot(h, W2[...], preferred_element_type=jnp.float32) + b2[...]
    h = _bn_relu(h, g2[...], be2[...])
    h = jnp.dot(h, W3[...], preferred_element_type=jnp.float32) + b3[...]
    h = _bn_relu(h, g3[...], be3[...])
    logits = jnp.dot(h, Wout[...], preferred_element_type=jnp.float32) + bout[...]
    logits_ref[...] = logits
    t = t_ref[...]
    per = (jnp.maximum(logits, 0.0) - logits * t
           + jnp.log1p(jnp.exp(-jnp.abs(logits))))
    loss_ref[...] = jnp.mean(per).reshape(1, 1)


_mlp = pl.pallas_call(
    _mlp_body,
    out_shape=(jax.ShapeDtypeStruct((1, 1), jnp.float32),
               jax.ShapeDtypeStruct((_B, 1), jnp.float32)),
)


def kernel(x, t, table, W1, b1, g1, be1, W2, b2, g2, be2,
           W3, b3, g3, be3, Wout, bout):
    xi = x.astype(jnp.int32)
    bow_sum = _bow_sum_sc(xi, table)
    loss2, logits2 = _mlp(
        bow_sum, t.reshape(_B, 1),
        W1, b1.reshape(1, _H), g1.reshape(1, _H), be1.reshape(1, _H),
        W2, b2.reshape(1, _H), g2.reshape(1, _H), be2.reshape(1, _H),
        W3, b3.reshape(1, _H), g3.reshape(1, _H), be3.reshape(1, _H),
        Wout, bout.reshape(1, 1))
    return loss2[0, 0], logits2[:, 0]


# speedup vs baseline: 1.4061x; 1.2708x over previous
import functools

import jax
import jax.numpy as jnp
from jax import lax
from jax.experimental import pallas as pl
from jax.experimental.pallas import tpu as pltpu
from jax.experimental.pallas import tpu_sc as plsc

_B, _L, _V, _H = 4096, 200, 1000000, 64
_NC, _NS = 2, 16
_NW = _NC * _NS
_BPW = _B // _NW
_C0 = 128
_C1 = _L - _C0
_LP = 208
_HALF = 1 << 19
_TCOLS = 2048
_TGRID = _HALF // _TCOLS

_mesh = plsc.VectorSubcoreMesh(core_axis_name="c", subcore_axis_name="s")


def _relayout_body(a_ref, b_ref, out_ref):
    out_ref[:, 0:_H] = a_ref[...].T
    out_ref[:, _H:2 * _H] = b_ref[...].T


_relayout = pl.pallas_call(
    _relayout_body,
    grid=(_TGRID,),
    in_specs=[pl.BlockSpec((_H, _TCOLS), lambda i: (0, i)),
              pl.BlockSpec((_H, _TCOLS),
                           lambda i: (0, jnp.minimum(i + _TGRID,
                                                     _V // _TCOLS)))],
    out_specs=pl.BlockSpec((_TCOLS, 2 * _H), lambda i: (i, 0)),
    out_shape=jax.ShapeDtypeStruct((_HALF, 2 * _H), jnp.float32),
)


@functools.partial(
    pl.kernel,
    out_type=jax.ShapeDtypeStruct((_B, _H), jnp.float32),
    mesh=_mesh,
    scratch_types=[
        pltpu.VMEM((_BPW, _LP), jnp.int32),
        pltpu.VMEM((2, _LP), jnp.int32),
        pltpu.VMEM((2, _LP, 2 * _H), jnp.float32),
        pltpu.VMEM((_BPW, _H), jnp.float32),
        pltpu.SemaphoreType.DMA,
        pltpu.SemaphoreType.DMA,
    ],
    compiler_params=pltpu.CompilerParams(use_tc_tiling_on_sc=False),
)
def _pair_gather_sc(x_hbm, tbl2_hbm, out_hbm,
                    xr_v, idx2_v, rows_v, acc_v, s0, s1):
    wid = lax.axis_index("s") * _NC + lax.axis_index("c")
    base = wid * _BPW
    sems = (s0, s1)
    zero = jnp.zeros((16,), jnp.float32)
    pltpu.sync_copy(x_hbm.at[pl.ds(base, _BPW), :], xr_v)

    for slot in range(2):
        @pl.loop(_L, _LP)
        def _z(l, _slot=slot):
            for k in range(8):
                rows_v[_slot, l, pl.ds(k * 16, 16)] = zero

    def fire(b, rslot):
        sem = sems[rslot]

        @pl.loop(0, 13)
        def _mk(c):
            xv = xr_v[b, pl.ds(c * 16, 16)]
            idx2_v[rslot, pl.ds(c * 16, 16)] = jnp.bitwise_and(xv, _HALF - 1)

        pltpu.async_copy(tbl2_hbm.at[idx2_v.at[rslot, pl.ds(0, _C0)]],
                         rows_v.at[rslot, pl.ds(0, _C0), :], sem)
        pltpu.async_copy(tbl2_hbm.at[idx2_v.at[rslot, pl.ds(_C0, _C1)]],
                         rows_v.at[rslot, pl.ds(_C0, _C1), :], sem)

    def drain(rslot):
        pltpu.make_async_copy(tbl2_hbm.at[pl.ds(0, _L), :],
                              rows_v.at[rslot, pl.ds(0, _L), :],
                              sems[rslot]).wait()

    def accum_store(b, aslot, arow):
        def chunk(c, accs):
            a0, a1, a2, a3 = accs
            xv = xr_v[b, pl.ds(c * 16, 16)]
            ov = lax.shift_left(lax.shift_right_logical(xv, 19), 6)
            for j in range(16):
                off = ov[j]
                l = c * 16 + j
                a0 = a0 + rows_v[aslot, l, pl.ds(off, 16)]
                a1 = a1 + rows_v[aslot, l, pl.ds(off + 16, 16)]
                a2 = a2 + rows_v[aslot, l, pl.ds(off + 32, 16)]
                a3 = a3 + rows_v[aslot, l, pl.ds(off + 48, 16)]
            return (a0, a1, a2, a3)

        a0, a1, a2, a3 = lax.fori_loop(0, 13, chunk, (zero,) * 4)
        acc_v[arow, pl.ds(0, 16)] = a0
        acc_v[arow, pl.ds(16, 16)] = a1
        acc_v[arow, pl.ds(32, 16)] = a2
        acc_v[arow, pl.ds(48, 16)] = a3

    fire(0, 0)

    @pl.loop(0, _BPW - 2, step=2)
    def _pair(b):
        drain(0)
        fire(b + 1, 1)
        accum_store(b, 0, b)
        drain(1)
        fire(b + 2, 0)
        accum_store(b + 1, 1, b + 1)

    drain(0)
    fire(_BPW - 1, 1)
    accum_store(_BPW - 2, 0, _BPW - 2)
    drain(1)
    accum_store(_BPW - 1, 1, _BPW - 1)
    pltpu.sync_copy(acc_v, out_hbm.at[pl.ds(base, _BPW), :])


def _bn_relu(h, g, be, eps=1e-5):
    mu = jnp.mean(h, axis=0, keepdims=True)
    var = jnp.mean((h - mu) * (h - mu), axis=0, keepdims=True)
    return jnp.maximum(g * (h - mu) / jnp.sqrt(var + eps) + be, 0.0)


def _mlp_body(bow_ref, t_ref, W1, b1, g1, be1, W2, b2, g2, be2,
              W3, b3, g3, be3, Wout, bout, loss_ref, logits_ref):
    bow = bow_ref[...] * (1.0 / _L)
    h = jnp.dot(bow, W1[...], preferred_element_type=jnp.float32) + b1[...]
    h = _bn_relu(h, g1[...], be1[...])
    h = jnp.dot(h, W2[...], preferred_element_type=jnp.float32) + b2[...]
    h = _bn_relu(h, g2[...], be2[...])
    h = jnp.dot(h, W3[...], preferred_element_type=jnp.float32) + b3[...]
    h = _bn_relu(h, g3[...], be3[...])
    logits = jnp.dot(h, Wout[...], preferred_element_type=jnp.float32) + bout[...]
    logits_ref[...] = logits
    t = t_ref[...]
    per = (jnp.maximum(logits, 0.0) - logits * t
           + jnp.log1p(jnp.exp(-jnp.abs(logits))))
    loss_ref[...] = jnp.mean(per).reshape(1, 1)


_mlp = pl.pallas_call(
    _mlp_body,
    out_shape=(jax.ShapeDtypeStruct((1, 1), jnp.float32),
               jax.ShapeDtypeStruct((_B, 1), jnp.float32)),
)


def kernel(x, t, table, W1, b1, g1, be1, W2, b2, g2, be2,
           W3, b3, g3, be3, Wout, bout):
    xi = x.astype(jnp.int32)
    xpad = jnp.pad(xi, ((0, 0), (0, _LP - _L)))
    tt = table.T
    tbl2 = _relayout(tt, tt)
    bow_sum = _pair_gather_sc(xpad, tbl2)
    loss2, logits2 = _mlp(
        bow_sum, t.reshape(_B, 1),
        W1, b1.reshape(1, _H), g1.reshape(1, _H), be1.reshape(1, _H),
        W2, b2.reshape(1, _H), g2.reshape(1, _H), be2.reshape(1, _H),
        W3, b3.reshape(1, _H), g3.reshape(1, _H), be3.reshape(1, _H),
        Wout, bout.reshape(1, 1))
    return loss2[0, 0], logits2[:, 0]


# speedup vs baseline: 1.6035x; 1.1404x over previous
import functools

import jax
import jax.numpy as jnp
from jax import lax
from jax.experimental import pallas as pl
from jax.experimental.pallas import tpu as pltpu
from jax.experimental.pallas import tpu_sc as plsc

_B, _L, _V, _H = 4096, 200, 1000000, 64
_NC, _NS = 2, 16
_NW = _NC * _NS
_BPW = _B // _NW
_C0 = 128
_C1 = _L - _C0
_LP = 208
_HALF = 1 << 19
_TCOLS = 2048
_TGRID = _HALF // _TCOLS

_mesh = plsc.VectorSubcoreMesh(core_axis_name="c", subcore_axis_name="s")


def _relayout_body(a_ref, b_ref, out_ref):
    out_ref[:, 0:_H] = a_ref[...].T
    out_ref[:, _H:2 * _H] = b_ref[...].T


_relayout = pl.pallas_call(
    _relayout_body,
    grid=(_TGRID,),
    in_specs=[pl.BlockSpec((_H, _TCOLS), lambda i: (0, i)),
              pl.BlockSpec((_H, _TCOLS),
                           lambda i: (0, jnp.minimum(i + _TGRID,
                                                     _V // _TCOLS)))],
    out_specs=pl.BlockSpec((_TCOLS, 2 * _H), lambda i: (i, 0)),
    out_shape=jax.ShapeDtypeStruct((_HALF, 2 * _H), jnp.float32),
)


@functools.partial(
    pl.kernel,
    out_type=jax.ShapeDtypeStruct((_B, _H), jnp.float32),
    mesh=_mesh,
    scratch_types=[
        pltpu.VMEM((_BPW, _LP), jnp.int32),
        pltpu.VMEM((2, _LP), jnp.int32),
        pltpu.VMEM((2, _L, _H), jnp.float32),
        pltpu.VMEM((_BPW, _H), jnp.float32),
        pltpu.SemaphoreType.DMA,
        pltpu.SemaphoreType.DMA,
    ],
    compiler_params=pltpu.CompilerParams(use_tc_tiling_on_sc=False),
)
def _pair_gather_sc(x_hbm, tbl3_hbm, out_hbm,
                    xr_v, idx3_v, rows_v, acc_v, s0, s1):
    wid = lax.axis_index("s") * _NC + lax.axis_index("c")
    base = wid * _BPW
    sems = (s0, s1)
    zero = jnp.zeros((16,), jnp.float32)
    pltpu.sync_copy(x_hbm.at[pl.ds(base, _BPW), :], xr_v)

    def fire(b, rslot):
        sem = sems[rslot]

        @pl.loop(0, 13)
        def _mk(c):
            xv = xr_v[b, pl.ds(c * 16, 16)]
            idx3_v[rslot, pl.ds(c * 16, 16)] = jnp.bitwise_or(
                lax.shift_left(jnp.bitwise_and(xv, _HALF - 1), 1),
                lax.shift_right_logical(xv, 19))

        pltpu.async_copy(tbl3_hbm.at[idx3_v.at[rslot, pl.ds(0, _C0)]],
                         rows_v.at[rslot, pl.ds(0, _C0), :], sem)
        pltpu.async_copy(tbl3_hbm.at[idx3_v.at[rslot, pl.ds(_C0, _C1)]],
                         rows_v.at[rslot, pl.ds(_C0, _C1), :], sem)

    def drain(rslot):
        pltpu.make_async_copy(tbl3_hbm.at[pl.ds(0, _L), :],
                              rows_v.at[rslot], sems[rslot]).wait()

    def accum_store(aslot, arow):
        def body(l, accs):
            a0, a1, a2, a3 = accs
            return (a0 + rows_v[aslot, l, pl.ds(0, 16)],
                    a1 + rows_v[aslot, l, pl.ds(16, 16)],
                    a2 + rows_v[aslot, l, pl.ds(32, 16)],
                    a3 + rows_v[aslot, l, pl.ds(48, 16)])

        a0, a1, a2, a3 = lax.fori_loop(0, _L, body, (zero,) * 4, unroll=8)
        acc_v[arow, pl.ds(0, 16)] = a0
        acc_v[arow, pl.ds(16, 16)] = a1
        acc_v[arow, pl.ds(32, 16)] = a2
        acc_v[arow, pl.ds(48, 16)] = a3

    fire(0, 0)

    @pl.loop(0, _BPW - 2, step=2)
    def _pair(b):
        drain(0)
        fire(b + 1, 1)
        accum_store(0, b)
        drain(1)
        fire(b + 2, 0)
        accum_store(1, b + 1)

    drain(0)
    fire(_BPW - 1, 1)
    accum_store(0, _BPW - 2)
    drain(1)
    accum_store(1, _BPW - 1)
    pltpu.sync_copy(acc_v, out_hbm.at[pl.ds(base, _BPW), :])


def _bn_relu(h, g, be, eps=1e-5):
    mu = jnp.mean(h, axis=0, keepdims=True)
    var = jnp.mean((h - mu) * (h - mu), axis=0, keepdims=True)
    return jnp.maximum(g * (h - mu) / jnp.sqrt(var + eps) + be, 0.0)


def _mlp_body(bow_ref, t_ref, W1, b1, g1, be1, W2, b2, g2, be2,
              W3, b3, g3, be3, Wout, bout, loss_ref, logits_ref):
    bow = bow_ref[...] * (1.0 / _L)
    h = jnp.dot(bow, W1[...], preferred_element_type=jnp.float32) + b1[...]
    h = _bn_relu(h, g1[...], be1[...])
    h = jnp.dot(h, W2[...], preferred_element_type=jnp.float32) + b2[...]
    h = _bn_relu(h, g2[...], be2[...])
    h = jnp.dot(h, W3[...], preferred_element_type=jnp.float32) + b3[...]
    h = _bn_relu(h, g3[...], be3[...])
    logits = jnp.dot(h, Wout[...], preferred_element_type=jnp.float32) + bout[...]
    logits_ref[...] = logits
    t = t_ref[...]
    per = (jnp.maximum(logits, 0.0) - logits * t
           + jnp.log1p(jnp.exp(-jnp.abs(logits))))
    loss_ref[...] = jnp.mean(per).reshape(1, 1)


_mlp = pl.pallas_call(
    _mlp_body,
    out_shape=(jax.ShapeDtypeStruct((1, 1), jnp.float32),
               jax.ShapeDtypeStruct((_B, 1), jnp.float32)),
)


def kernel(x, t, table, W1, b1, g1, be1, W2, b2, g2, be2,
           W3, b3, g3, be3, Wout, bout):
    xi = x.astype(jnp.int32)
    xpad = jnp.pad(xi, ((0, 0), (0, _LP - _L)))
    tt = table.T
    tbl2 = _relayout(tt, tt)
    bow_sum = _pair_gather_sc(xpad, tbl2.reshape(2 * _HALF, _H))
    loss2, logits2 = _mlp(
        bow_sum, t.reshape(_B, 1),
        W1, b1.reshape(1, _H), g1.reshape(1, _H), be1.reshape(1, _H),
        W2, b2.reshape(1, _H), g2.reshape(1, _H), be2.reshape(1, _H),
        W3, b3.reshape(1, _H), g3.reshape(1, _H), be3.reshape(1, _H),
        Wout, bout.reshape(1, 1))
    return loss2[0, 0], logits2[:, 0]


# speedup vs baseline: 1.8552x; 1.1570x over previous
import functools

import jax
import jax.numpy as jnp
from jax import lax
from jax.experimental import pallas as pl
from jax.experimental.pallas import tpu as pltpu
from jax.experimental.pallas import tpu_sc as plsc

_B, _L, _V, _H = 4096, 200, 1000000, 64
_NC, _NS = 2, 16
_NW = _NC * _NS
_BPW = _B // _NW
_C0 = 128
_C1 = _L - _C0
_LP = 208
_HALF = 1 << 19
_TCOLS = 2048
_TGRID = _HALF // _TCOLS

_mesh = plsc.VectorSubcoreMesh(core_axis_name="c", subcore_axis_name="s")


def _relayout_body(a_ref, b_ref, out_ref):
    out_ref[:, 0:_H] = a_ref[...].T
    out_ref[:, _H:2 * _H] = b_ref[...].T


_relayout = pl.pallas_call(
    _relayout_body,
    grid=(_TGRID,),
    in_specs=[pl.BlockSpec((_H, _TCOLS), lambda i: (0, i)),
              pl.BlockSpec((_H, _TCOLS),
                           lambda i: (0, jnp.minimum(i + _TGRID,
                                                     _V // _TCOLS)))],
    out_specs=pl.BlockSpec((_TCOLS, 2 * _H), lambda i: (i, 0)),
    out_shape=jax.ShapeDtypeStruct((_HALF, 2 * _H), jnp.float32),
)


@functools.partial(
    pl.kernel,
    out_type=jax.ShapeDtypeStruct((_B, _H), jnp.float32),
    mesh=_mesh,
    scratch_types=[
        pltpu.VMEM((_BPW, _L), jnp.int32),
        pltpu.VMEM((3, _LP), jnp.int32),
        pltpu.VMEM((3, _L, _H), jnp.float32),
        pltpu.VMEM((_BPW, _H), jnp.float32),
        pltpu.SemaphoreType.DMA,
        pltpu.SemaphoreType.DMA,
        pltpu.SemaphoreType.DMA,
    ],
    compiler_params=pltpu.CompilerParams(use_tc_tiling_on_sc=False),
)
def _pair_gather_sc(x_hbm, tbl3_hbm, out_hbm,
                    xr_v, idx3_v, rows_v, acc_v, s0, s1, s2):
    wid = lax.axis_index("s") * _NC + lax.axis_index("c")
    base = wid * _BPW
    sems = (s0, s1, s2)
    zero = jnp.zeros((16,), jnp.float32)
    pltpu.sync_copy(x_hbm.at[pl.ds(base, _BPW), :], xr_v)

    def fire(b, rslot):
        sem = sems[rslot]

        @pl.loop(0, 13)
        def _mk(c):
            o = jnp.minimum(c * 16, _L - 16)
            xv = xr_v[b, pl.ds(o, 16)]
            idx3_v[rslot, pl.ds(o, 16)] = jnp.bitwise_or(
                lax.shift_left(jnp.bitwise_and(xv, _HALF - 1), 1),
                lax.shift_right_logical(xv, 19))

        pltpu.async_copy(tbl3_hbm.at[idx3_v.at[rslot, pl.ds(0, _C0)]],
                         rows_v.at[rslot, pl.ds(0, _C0), :], sem)
        pltpu.async_copy(tbl3_hbm.at[idx3_v.at[rslot, pl.ds(_C0, _C1)]],
                         rows_v.at[rslot, pl.ds(_C0, _C1), :], sem)

    def drain(rslot):
        pltpu.make_async_copy(tbl3_hbm.at[pl.ds(0, _L), :],
                              rows_v.at[rslot], sems[rslot]).wait()

    def accum_store(aslot, arow):
        def body(l, accs):
            a0, a1, a2, a3 = accs
            return (a0 + rows_v[aslot, l, pl.ds(0, 16)],
                    a1 + rows_v[aslot, l, pl.ds(16, 16)],
                    a2 + rows_v[aslot, l, pl.ds(32, 16)],
                    a3 + rows_v[aslot, l, pl.ds(48, 16)])

        a0, a1, a2, a3 = lax.fori_loop(0, _L, body, (zero,) * 4, unroll=8)
        acc_v[arow, pl.ds(0, 16)] = a0
        acc_v[arow, pl.ds(16, 16)] = a1
        acc_v[arow, pl.ds(32, 16)] = a2
        acc_v[arow, pl.ds(48, 16)] = a3

    fire(0, 0)
    fire(1, 1)

    @pl.loop(0, _BPW - 2, step=3)
    def _trip(b):
        drain(0)
        fire(b + 2, 2)
        accum_store(0, b)
        drain(1)
        fire(b + 3, 0)
        accum_store(1, b + 1)
        drain(2)
        fire(b + 4, 1)
        accum_store(2, b + 2)

    drain(0)
    accum_store(0, _BPW - 2)
    drain(1)
    accum_store(1, _BPW - 1)
    pltpu.sync_copy(acc_v, out_hbm.at[pl.ds(base, _BPW), :])


def _bn_relu(h, g, be, eps=1e-5):
    mu = jnp.mean(h, axis=0, keepdims=True)
    var = jnp.mean((h - mu) * (h - mu), axis=0, keepdims=True)
    return jnp.maximum(g * (h - mu) / jnp.sqrt(var + eps) + be, 0.0)


def _mlp_body(bow_ref, t_ref, W1, b1, g1, be1, W2, b2, g2, be2,
              W3, b3, g3, be3, Wout, bout, loss_ref, logits_ref):
    bow = bow_ref[...] * (1.0 / _L)
    h = jnp.dot(bow, W1[...], preferred_element_type=jnp.float32) + b1[...]
    h = _bn_relu(h, g1[...], be1[...])
    h = jnp.dot(h, W2[...], preferred_element_type=jnp.float32) + b2[...]
    h = _bn_relu(h, g2[...], be2[...])
    h = jnp.dot(h, W3[...], preferred_element_type=jnp.float32) + b3[...]
    h = _bn_relu(h, g3[...], be3[...])
    logits = jnp.dot(h, Wout[...], preferred_element_type=jnp.float32) + bout[...]
    logits_ref[...] = logits
    t = t_ref[...]
    per = (jnp.maximum(logits, 0.0) - logits * t
           + jnp.log1p(jnp.exp(-jnp.abs(logits))))
    loss_ref[...] = jnp.mean(per).reshape(1, 1)


_mlp = pl.pallas_call(
    _mlp_body,
    out_shape=(jax.ShapeDtypeStruct((1, 1), jnp.float32),
               jax.ShapeDtypeStruct((_B, 1), jnp.float32)),
)


def kernel(x, t, table, W1, b1, g1, be1, W2, b2, g2, be2,
           W3, b3, g3, be3, Wout, bout):
    xi = x.astype(jnp.int32)
    tt = table.T
    tbl2 = _relayout(tt, tt)
    bow_sum = _pair_gather_sc(xi, tbl2.reshape(2 * _HALF, _H))
    loss2, logits2 = _mlp(
        bow_sum, t.reshape(_B, 1),
        W1, b1.reshape(1, _H), g1.reshape(1, _H), be1.reshape(1, _H),
        W2, b2.reshape(1, _H), g2.reshape(1, _H), be2.reshape(1, _H),
        W3, b3.reshape(1, _H), g3.reshape(1, _H), be3.reshape(1, _H),
        Wout, bout.reshape(1, 1))
    return loss2[0, 0], logits2[:, 0]
